# baseline (speedup 1.0000x reference)
import functools

import jax
import jax.numpy as jnp
from jax.experimental import pallas as pl
from jax.experimental.pallas import tpu as pltpu

_EPS = 1e-5


def _softplus(x):
    return jnp.maximum(x, 0.0) + jnp.log1p(jnp.exp(-jnp.abs(x)))


def _edge_body(nedges, xrow, xcol, nbr, w1, w2, wn, bfc, g1, b1,
               msg_out, stats):
    p = pl.program_id(0)
    j = pl.program_id(1)

    @pl.when((p == 0) & (j == 0))
    def _init():
        stats[...] = jnp.zeros_like(stats)

    gated = (
        jnp.dot(xrow[...], w1[...], preferred_element_type=jnp.float32)
        + jnp.dot(xcol[...], w2[...], preferred_element_type=jnp.float32)
        + jnp.dot(nbr[...], wn[...], preferred_element_type=jnp.float32)
        + bfc[...]
    )

    @pl.when(p == 0)
    def _accum():
        stats[0:1, :] += jnp.sum(gated, axis=0, keepdims=True)
        stats[1:2, :] += jnp.sum(gated * gated, axis=0, keepdims=True)

    @pl.when(p == 1)
    def _apply():
        @pl.when(j == 0)
        def _finalize():
            mean = stats[0:1, :] / nedges
            var = stats[1:2, :] / nedges - mean * mean
            scale = g1[...] / jnp.sqrt(var + _EPS)
            stats[2:3, :] = scale
            stats[3:4, :] = b1[...] - mean * scale

        z = gated * stats[2:3, :] + stats[3:4, :]
        half = z.shape[1] // 2
        filt = z[:, :half]
        core = z[:, half:]
        msg_out[...] = jax.nn.sigmoid(filt) * _softplus(core)


def _edge_messages(xrow_g, xcol_g, nbr_fea, wfc, bfc, g1, b1):
    e, afl = xrow_g.shape
    nbrl = nbr_fea.shape[1]
    two_afl = 2 * afl
    eblk = 4000 if e % 4000 == 0 else e
    nb = e // eblk

    w1 = wfc[:afl]
    w2 = wfc[afl:two_afl]
    wn = wfc[two_afl:]

    grid = (2, nb)
    bs_edge = lambda width: pl.BlockSpec((eblk, width), lambda p, j: (j, 0))
    bs_full = lambda a, b: pl.BlockSpec((a, b), lambda p, j: (0, 0))

    out = pl.pallas_call(
        functools.partial(_edge_body, float(e)),
        grid=grid,
        in_specs=[
            bs_edge(afl),
            bs_edge(afl),
            bs_edge(nbrl),
            bs_full(afl, two_afl),
            bs_full(afl, two_afl),
            bs_full(nbrl, two_afl),
            bs_full(1, two_afl),
            bs_full(1, two_afl),
            bs_full(1, two_afl),
        ],
        out_specs=pl.BlockSpec((eblk, afl),
                               lambda p, j: (jnp.where(p == 1, j, 0), 0)),
        out_shape=jax.ShapeDtypeStruct((e, afl), jnp.float32),
        scratch_shapes=[pltpu.VMEM((8, two_afl), jnp.float32)],
    )(xrow_g, xcol_g, nbr_fea, w1, w2, wn,
      bfc.reshape(1, -1), g1.reshape(1, -1), b1.reshape(1, -1))
    return out


def kernel(atom_fea, nbr_fea, nbr_fea_idx, dists, crystal_atom_idx, batch,
           W_emb, b_emb, Wfc, bfc, g1, b1, g2, b2, W_c2f, b_c2f, W_out, b_out):
    n = atom_fea.shape[0]
    ncrys = 256

    x = atom_fea @ W_emb + b_emb
    row = nbr_fea_idx[0]
    col = nbr_fea_idx[1]

    for i in range(Wfc.shape[0]):
        xrow_g = jnp.take(x, row, axis=0)
        xcol_g = jnp.take(x, col, axis=0)
        msg = _edge_messages(xrow_g, xcol_g, nbr_fea, Wfc[i], bfc[i],
                             g1[i], b1[i])
        summed = jax.ops.segment_sum(msg, col, num_segments=n)
        m = jnp.mean(summed, axis=0, keepdims=True)
        v = jnp.var(summed, axis=0, keepdims=True)
        summed = g2[i] * (summed - m) / jnp.sqrt(v + _EPS) + b2[i]
        x = _softplus(x + summed)

    sums = jax.ops.segment_sum(x, batch, num_segments=ncrys)
    counts = jax.ops.segment_sum(jnp.ones((n, 1), x.dtype), batch,
                                 num_segments=ncrys)
    crys = sums / jnp.maximum(counts, 1.0)
    crys = _softplus(crys) @ W_c2f + b_c2f
    crys = _softplus(crys)
    return crys @ W_out + b_out
